# Initial kernel scaffold; baseline (speedup 1.0000x reference)
import jax
import jax.numpy as jnp
from jax.experimental import pallas as pl


def kernel(x, plane):
    raise NotImplementedError("write your pallas kernel here")


# speedup vs baseline: 1.0619x; 1.0619x over previous
import functools

import jax
import jax.numpy as jnp
from jax import lax
from jax.experimental import pallas as pl
from jax.experimental.pallas import tpu as pltpu
from jax.experimental.pallas import tpu_sc as plsc

_W, _H, _D = 1024, 1024, 64
_N = 524288
_NC = 2
_NS = 16
_NW = _NC * _NS
_PW = _N // _NW
_C = 128
_NCH = _PW // _C
_L = 16
_G = _C // _L


def _prepare(chunk, wid, x_hbm, plane_hbm, xb, idxb, wb, rows, gsem):
    base = wid * _PW + chunk * _C
    pltpu.sync_copy(x_hbm.at[pl.ds(base * 2, _C * 2)], xb)
    lane = lax.iota(jnp.int32, _L)
    for g in range(_G):
        row2 = (lane + g * _L) * 2
        xs = plsc.load_gather(xb, [row2])
        ys = plsc.load_gather(xb, [row2 + 1])
        x0 = xs.astype(jnp.int32)
        y0 = ys.astype(jnp.int32)
        tx = xs - x0.astype(jnp.float32)
        ty = ys - y0.astype(jnp.float32)
        i00 = x0 * _H + y0
        sl = pl.ds(g * _L, _L)
        idxb[0, sl] = i00
        idxb[1, sl] = i00 + _H
        idxb[2, sl] = i00 + 1
        idxb[3, sl] = i00 + _H + 1
        u = 1.0 - tx
        v = 1.0 - ty
        wsl = g * _L
        wb[pl.ds(0 * _C + wsl, _L)] = u * v
        wb[pl.ds(1 * _C + wsl, _L)] = tx * v
        wb[pl.ds(2 * _C + wsl, _L)] = u * ty
        wb[pl.ds(3 * _C + wsl, _L)] = tx * ty
    for c in range(4):
        pltpu.async_copy(plane_hbm.at[idxb.at[c]], rows.at[c], gsem)


def _compute(chunk, wid, plane_hbm, out_hbm, idxb, wb, rows, ob, gsem):
    for c in range(4):
        pltpu.make_async_copy(plane_hbm.at[idxb.at[c]], rows.at[c], gsem).wait()

    zeros = jnp.zeros((_L,), jnp.int32)

    def body(i, carry):
        iv = zeros + i
        w00 = plsc.load_gather(wb, [iv])
        w10 = plsc.load_gather(wb, [iv + _C])
        w01 = plsc.load_gather(wb, [iv + 2 * _C])
        w11 = plsc.load_gather(wb, [iv + 3 * _C])
        for k in range(_D // _L):
            sl = pl.ds(k * _L, _L)
            ob[i, sl] = (rows[0, i, sl] * w00 + rows[1, i, sl] * w10
                         + rows[2, i, sl] * w01 + rows[3, i, sl] * w11)
        return carry

    lax.fori_loop(0, _C, body, 0)
    base = wid * _PW + chunk * _C
    pltpu.sync_copy(ob, out_hbm.at[pl.ds(base, _C)])


@functools.partial(
    pl.kernel,
    out_type=jax.ShapeDtypeStruct((_N, _D), jnp.float32),
    mesh=plsc.VectorSubcoreMesh(core_axis_name="c", subcore_axis_name="s"),
    compiler_params=pltpu.CompilerParams(
        needs_layout_passes=False, use_tc_tiling_on_sc=False),
    scratch_types=[
        pltpu.VMEM((_C * 2,), jnp.float32),
        pltpu.VMEM((4, _C), jnp.int32),
        pltpu.VMEM((4 * _C,), jnp.float32),
        pltpu.VMEM((4, _C, _D), jnp.float32),
        pltpu.VMEM((_C, _D), jnp.float32),
        pltpu.SemaphoreType.DMA,
        pltpu.VMEM((_C * 2,), jnp.float32),
        pltpu.VMEM((4, _C), jnp.int32),
        pltpu.VMEM((4 * _C,), jnp.float32),
        pltpu.VMEM((4, _C, _D), jnp.float32),
        pltpu.VMEM((_C, _D), jnp.float32),
        pltpu.SemaphoreType.DMA,
    ],
)
def _bilerp_sc(x_hbm, plane_hbm, out_hbm,
               xbA, idxA, wbA, rowsA, obA, gsemA,
               xbB, idxB, wbB, rowsB, obB, gsemB):
    wid = lax.axis_index("s") * _NC + lax.axis_index("c")
    _prepare(0, wid, x_hbm, plane_hbm, xbA, idxA, wbA, rowsA, gsemA)

    def pair(p, carry):
        g = p * 2
        _prepare(g + 1, wid, x_hbm, plane_hbm, xbB, idxB, wbB, rowsB, gsemB)
        _compute(g, wid, plane_hbm, out_hbm, idxA, wbA, rowsA, obA, gsemA)

        @pl.when(g + 2 < _NCH)
        def _():
            _prepare(g + 2, wid, x_hbm, plane_hbm, xbA, idxA, wbA, rowsA,
                     gsemA)

        _compute(g + 1, wid, plane_hbm, out_hbm, idxB, wbB, rowsB, obB, gsemB)
        return carry

    lax.fori_loop(0, _NCH // 2, pair, 0)


def kernel(x, plane):
    return _bilerp_sc(x.reshape(_N * 2), plane.reshape(_W * _H, _D))


# speedup vs baseline: 1.0727x; 1.0102x over previous
import functools

import jax
import jax.numpy as jnp
from jax import lax
from jax.experimental import pallas as pl
from jax.experimental.pallas import tpu as pltpu
from jax.experimental.pallas import tpu_sc as plsc

_W, _H, _D = 1024, 1024, 64
_N = 524288
_NC = 2
_NS = 16
_NW = _NC * _NS
_PW = _N // _NW
_C = 128
_NCH = _PW // _C
_L = 16
_G = _C // _L


def _prepare(chunk, wid, x_hbm, plane_hbm, xb, idxb, wb, rows, gsem):
    base = wid * _PW + chunk * _C
    pltpu.sync_copy(x_hbm.at[pl.ds(base * 2, _C * 2)], xb)
    lane = lax.iota(jnp.int32, _L)
    for g in range(_G):
        row2 = (lane + g * _L) * 2
        xs = plsc.load_gather(xb, [row2])
        ys = plsc.load_gather(xb, [row2 + 1])
        x0 = xs.astype(jnp.int32)
        y0 = ys.astype(jnp.int32)
        tx = xs - x0.astype(jnp.float32)
        ty = ys - y0.astype(jnp.float32)
        i00 = x0 * _H + y0
        sl = pl.ds(g * _L, _L)
        idxb[0, sl] = i00
        idxb[1, sl] = i00 + _H
        idxb[2, sl] = i00 + 1
        idxb[3, sl] = i00 + _H + 1
        wsl = g * _L
        wb[pl.ds(wsl, _L)] = tx
        wb[pl.ds(_C + wsl, _L)] = ty
    for c in range(4):
        pltpu.async_copy(plane_hbm.at[idxb.at[c]], rows.at[c], gsem)


def _compute(chunk, wid, plane_hbm, out_hbm, idxb, wb, rows, ob, gsem):
    for c in range(4):
        pltpu.make_async_copy(plane_hbm.at[idxb.at[c]], rows.at[c], gsem).wait()

    zeros = jnp.zeros((_L,), jnp.int32)

    @plsc.parallel_loop(0, _C, unroll=4)
    def body(i):
        iv = zeros + i
        txv = plsc.load_gather(wb, [iv])
        tyv = plsc.load_gather(wb, [iv + _C])
        for k in range(_D // _L):
            sl = pl.ds(k * _L, _L)
            p00 = rows[0, i, sl]
            p10 = rows[1, i, sl]
            p01 = rows[2, i, sl]
            p11 = rows[3, i, sl]
            top = p00 + txv * (p10 - p00)
            bot = p01 + txv * (p11 - p01)
            ob[i, sl] = top + tyv * (bot - top)
        return
    base = wid * _PW + chunk * _C
    pltpu.sync_copy(ob, out_hbm.at[pl.ds(base, _C)])


@functools.partial(
    pl.kernel,
    out_type=jax.ShapeDtypeStruct((_N, _D), jnp.float32),
    mesh=plsc.VectorSubcoreMesh(core_axis_name="c", subcore_axis_name="s"),
    compiler_params=pltpu.CompilerParams(
        needs_layout_passes=False, use_tc_tiling_on_sc=False),
    scratch_types=[
        pltpu.VMEM((_C * 2,), jnp.float32),
        pltpu.VMEM((4, _C), jnp.int32),
        pltpu.VMEM((4 * _C,), jnp.float32),
        pltpu.VMEM((4, _C, _D), jnp.float32),
        pltpu.VMEM((_C, _D), jnp.float32),
        pltpu.SemaphoreType.DMA,
        pltpu.VMEM((_C * 2,), jnp.float32),
        pltpu.VMEM((4, _C), jnp.int32),
        pltpu.VMEM((4 * _C,), jnp.float32),
        pltpu.VMEM((4, _C, _D), jnp.float32),
        pltpu.VMEM((_C, _D), jnp.float32),
        pltpu.SemaphoreType.DMA,
    ],
)
def _bilerp_sc(x_hbm, plane_hbm, out_hbm,
               xbA, idxA, wbA, rowsA, obA, gsemA,
               xbB, idxB, wbB, rowsB, obB, gsemB):
    wid = lax.axis_index("s") * _NC + lax.axis_index("c")
    _prepare(0, wid, x_hbm, plane_hbm, xbA, idxA, wbA, rowsA, gsemA)

    def pair(p, carry):
        g = p * 2
        _prepare(g + 1, wid, x_hbm, plane_hbm, xbB, idxB, wbB, rowsB, gsemB)
        _compute(g, wid, plane_hbm, out_hbm, idxA, wbA, rowsA, obA, gsemA)

        @pl.when(g + 2 < _NCH)
        def _():
            _prepare(g + 2, wid, x_hbm, plane_hbm, xbA, idxA, wbA, rowsA,
                     gsemA)

        _compute(g + 1, wid, plane_hbm, out_hbm, idxB, wbB, rowsB, obB, gsemB)
        return carry

    lax.fori_loop(0, _NCH // 2, pair, 0)


def kernel(x, plane):
    return _bilerp_sc(x.reshape(_N * 2), plane.reshape(_W * _H, _D))
